# baseline (speedup 1.0000x reference)
import functools

import jax
import jax.numpy as jnp
from jax import lax
from jax.experimental import pallas as pl
from jax.experimental.pallas import tpu as pltpu
from jax.experimental.pallas import tpu_sc as plsc

D_MODEL = 128
SCALE = float(D_MODEL) ** 0.5
LANES = 16
NUM_CORES = 2
NUM_SUBCORES = 16
NUM_WORKERS = NUM_CORES * NUM_SUBCORES
CHUNK = 128
NBUF = 7


def _make_lookup(batch: int, row_len: int):
    assert batch % (NUM_WORKERS * CHUNK) == 0
    per_worker = batch // NUM_WORKERS
    n_chunks = per_worker // CHUNK
    assert row_len % per_worker == 0

    mesh = plsc.VectorSubcoreMesh(core_axis_name="c", subcore_axis_name="s")

    @functools.partial(
        pl.kernel,
        mesh=mesh,
        out_type=jax.ShapeDtypeStruct((batch, D_MODEL), jnp.float32),
        scratch_types=[
            pltpu.VMEM((per_worker,), jnp.int32),
            pltpu.VMEM((NBUF, CHUNK, D_MODEL), jnp.float32),
        ]
        + [pltpu.SemaphoreType.DMA] * (2 * NBUF),
    )
    def lookup(ids_hbm, table_hbm, out_hbm, idx_v, rows_v, *sems):
        gsem = sems[:NBUF]
        ssem = sems[NBUF:]
        wid = lax.axis_index("s") * NUM_CORES + lax.axis_index("c")
        base = wid * per_worker
        row = wid // (row_len // per_worker)
        col = (wid % (row_len // per_worker)) * per_worker
        pltpu.sync_copy(
            ids_hbm.at[row, pl.ds(col, CHUNK)], idx_v.at[pl.ds(0, CHUNK)]
        )

        def start_gather(c):
            b = c % NBUF
            return pltpu.async_copy(
                table_hbm.at[idx_v.at[pl.ds(c * CHUNK, CHUNK)]], rows_v.at[b], gsem[b]
            )

        gathers = [None] * n_chunks
        scatters = [None] * n_chunks
        gathers[0] = start_gather(0)
        if n_chunks > 1:
            pltpu.sync_copy(
                ids_hbm.at[row, pl.ds(col + CHUNK, per_worker - CHUNK)],
                idx_v.at[pl.ds(CHUNK, per_worker - CHUNK)],
            )
        for c in range(1, min(NBUF - 1, n_chunks)):
            gathers[c] = start_gather(c)
        for c in range(n_chunks):
            b = c % NBUF
            if c + NBUF - 1 < n_chunks:
                if c >= 1:
                    for s in scatters[c - 1]:
                        s.wait()
                gathers[c + NBUF - 1] = start_gather(c + NBUF - 1)
            gathers[c].wait()

            def scale_row(r, _):
                for j in range(D_MODEL // LANES):
                    sl = pl.ds(j * LANES, LANES)
                    rows_v[b, r, sl] = rows_v[b, r, sl] * SCALE
                return 0

            half = CHUNK // 2
            lax.fori_loop(0, half, scale_row, 0)
            s1 = pltpu.async_copy(
                rows_v.at[b, pl.ds(0, half)],
                out_hbm.at[pl.ds(base + c * CHUNK, half)],
                ssem[b],
            )
            lax.fori_loop(half, CHUNK, scale_row, 0)
            s2 = pltpu.async_copy(
                rows_v.at[b, pl.ds(half, CHUNK - half)],
                out_hbm.at[pl.ds(base + c * CHUNK + half, CHUNK - half)],
                ssem[b],
            )
            scatters[c] = (s1, s2)
        for c in range(max(0, n_chunks - NBUF), n_chunks):
            for s in scatters[c]:
                s.wait()

    return lookup


def kernel(input_ids, table):
    b0, b1 = input_ids.shape
    batch = b0 * b1
    out = _make_lookup(batch, b1)(input_ids, table)
    return out.reshape(b0, b1, D_MODEL)


# speedup vs baseline: 1.0140x; 1.0140x over previous
import functools

import jax
import jax.numpy as jnp
from jax import lax
from jax.experimental import pallas as pl
from jax.experimental.pallas import tpu as pltpu
from jax.experimental.pallas import tpu_sc as plsc

D_MODEL = 128
SCALE = float(D_MODEL) ** 0.5
LANES = 16
NUM_CORES = 2
NUM_SUBCORES = 16
NUM_WORKERS = NUM_CORES * NUM_SUBCORES
CHUNK = 128
NBUF = 6


def _make_lookup(batch: int, row_len: int):
    assert batch % (NUM_WORKERS * CHUNK) == 0
    per_worker = batch // NUM_WORKERS
    n_chunks = per_worker // CHUNK
    assert row_len % per_worker == 0

    mesh = plsc.VectorSubcoreMesh(core_axis_name="c", subcore_axis_name="s")

    @functools.partial(
        pl.kernel,
        mesh=mesh,
        out_type=jax.ShapeDtypeStruct((batch, D_MODEL), jnp.float32),
        scratch_types=[
            pltpu.VMEM((per_worker,), jnp.int32),
            pltpu.VMEM((NBUF, CHUNK, D_MODEL), jnp.float32),
        ]
        + [pltpu.SemaphoreType.DMA] * (2 * NBUF),
    )
    def lookup(ids_hbm, table_hbm, out_hbm, idx_v, rows_v, *sems):
        gsem = sems[:NBUF]
        ssem = sems[NBUF:]
        wid = lax.axis_index("s") * NUM_CORES + lax.axis_index("c")
        base = wid * per_worker
        row = wid // (row_len // per_worker)
        col = (wid % (row_len // per_worker)) * per_worker
        pltpu.sync_copy(
            ids_hbm.at[row, pl.ds(col, CHUNK)], idx_v.at[pl.ds(0, CHUNK)]
        )

        def start_gather(c):
            b = c % NBUF
            return pltpu.async_copy(
                table_hbm.at[idx_v.at[pl.ds(c * CHUNK, CHUNK)]], rows_v.at[b], gsem[b]
            )

        gathers = [None] * n_chunks
        scatters = [None] * n_chunks
        gathers[0] = start_gather(0)
        if n_chunks > 1:
            pltpu.sync_copy(
                ids_hbm.at[row, pl.ds(col + CHUNK, per_worker - CHUNK)],
                idx_v.at[pl.ds(CHUNK, per_worker - CHUNK)],
            )
        for c in range(1, min(NBUF - 1, n_chunks)):
            gathers[c] = start_gather(c)
        for c in range(n_chunks):
            b = c % NBUF
            if c + NBUF - 1 < n_chunks:
                if c >= 1:
                    for s in scatters[c - 1]:
                        s.wait()
                gathers[c + NBUF - 1] = start_gather(c + NBUF - 1)
            gathers[c].wait()

            def scale_row(r, _):
                for j in range(D_MODEL // LANES):
                    sl = pl.ds(j * LANES, LANES)
                    rows_v[b, r, sl] = rows_v[b, r, sl] * SCALE
                return 0

            half = CHUNK // 2
            lax.fori_loop(0, half, scale_row, 0)
            s1 = pltpu.async_copy(
                rows_v.at[b, pl.ds(0, half)],
                out_hbm.at[pl.ds(base + c * CHUNK, half)],
                ssem[b],
            )
            lax.fori_loop(half, CHUNK, scale_row, 0)
            s2 = pltpu.async_copy(
                rows_v.at[b, pl.ds(half, CHUNK - half)],
                out_hbm.at[pl.ds(base + c * CHUNK + half, CHUNK - half)],
                ssem[b],
            )
            scatters[c] = (s1, s2)
        for c in range(max(0, n_chunks - NBUF), n_chunks):
            for s in scatters[c]:
                s.wait()

    return lookup


def kernel(input_ids, table):
    b0, b1 = input_ids.shape
    batch = b0 * b1
    out = _make_lookup(batch, b1)(input_ids, table)
    return out.reshape(b0, b1, D_MODEL)
